# baseline (speedup 1.0000x reference)
import functools

import jax
import jax.numpy as jnp
from jax import lax
from jax.experimental import pallas as pl
from jax.experimental.pallas import tpu as pltpu
from jax.experimental.pallas import tpu_sc as plsc

BS, NV, DIM, NC = 8, 4096, 256, 10
NWORK = 32
RPW = NV // NWORK
NSEG = BS * NC
KCH = DIM // 16


def _sc_partial_sums(embeddings, meta):
    mesh = plsc.VectorSubcoreMesh(
        core_axis_name="c", subcore_axis_name="s",
        num_cores=2, num_subcores=16)

    @functools.partial(
        pl.kernel,
        out_type=(
            jax.ShapeDtypeStruct((NWORK, NSEG, DIM), jnp.float32),
            jax.ShapeDtypeStruct((NWORK, NSEG * 16), jnp.float32),
        ),
        mesh=mesh,
        scratch_types=[
            pltpu.VMEM((2 * NSEG + 16,), jnp.int32),
            pltpu.VMEM((RPW, DIM), jnp.float32),
            pltpu.VMEM((RPW, DIM), jnp.float32),
            pltpu.VMEM(((RPW + 1) * DIM,), jnp.float32),
            pltpu.VMEM(((RPW + 1) * 16,), jnp.float32),
            pltpu.VMEM((NSEG, DIM), jnp.float32),
            pltpu.VMEM((NSEG * 16,), jnp.float32),
            pltpu.SemaphoreType.DMA,
            pltpu.SemaphoreType.DMA,
        ],
    )
    def k(emb_hbm, meta_hbm, sum_out, sq_out,
          meta_v, chunk0_v, chunk1_v, pref_v, prefsq_v, acc_v, sq_v,
          sem0, sem1):
        wid = lax.axis_index("s") * 2 + lax.axis_index("c")
        base = wid * RPW
        pltpu.sync_copy(meta_hbm, meta_v)
        chunks = (chunk0_v, chunk1_v)
        sems = (sem0, sem1)

        copies = [None, None]
        copies[0] = pltpu.async_copy(
            emb_hbm.at[0, pl.ds(base, RPW), :], chunks[0], sems[0])
        for b in range(BS):
            cur = b % 2
            copies[cur].wait()
            if b + 1 < BS:
                nxt = (b + 1) % 2
                copies[nxt] = pltpu.async_copy(
                    emb_hbm.at[b + 1, pl.ds(base, RPW), :], chunks[nxt],
                    sems[nxt])
            chunk_v = chunks[cur]

            def pref_body(p, carry, chunk_v=chunk_v):
                accs, sq = carry
                na = []
                for kk in range(KCH):
                    pref_v[pl.ds(p * DIM + kk * 16, 16)] = accs[kk]
                    v = chunk_v[p, pl.ds(kk * 16, 16)]
                    na.append(accs[kk] + v)
                    sq = sq + v * v
                prefsq_v[pl.ds(p * 16, 16)] = carry[1]
                return (tuple(na), sq)

            z = tuple(jnp.zeros((16,), jnp.float32) for _ in range(KCH))
            accs, sq = lax.fori_loop(0, RPW, pref_body,
                                     (z, jnp.zeros((16,), jnp.float32)),
                                     unroll=4)
            for kk in range(KCH):
                pref_v[pl.ds(RPW * DIM + kk * 16, 16)] = accs[kk]
            prefsq_v[pl.ds(RPW * 16, 16)] = sq

            def cluster_body(j, _, b=b):
                s = meta_v[pl.ds(b * NC + j, 16)][0]
                e = meta_v[pl.ds(NSEG + b * NC + j, 16)][0]
                lo = jnp.minimum(jnp.maximum(s - base, 0), RPW)
                hi = jnp.minimum(jnp.maximum(e - base, 0), RPW)
                seg = b * NC + j
                for kk in range(KCH):
                    acc_v[seg, pl.ds(kk * 16, 16)] = (
                        pref_v[pl.ds(hi * DIM + kk * 16, 16)]
                        - pref_v[pl.ds(lo * DIM + kk * 16, 16)])
                sq_v[pl.ds(seg * 16, 16)] = (prefsq_v[pl.ds(hi * 16, 16)]
                    - prefsq_v[pl.ds(lo * 16, 16)])
                return 0

            lax.fori_loop(0, NC, cluster_body, 0)

        pltpu.sync_copy(acc_v, sum_out.at[wid])
        pltpu.sync_copy(sq_v, sq_out.at[wid])

    return k(embeddings, meta)


def _tc_finish(sum_parts, sq_parts, nf):
    def body(sum_ref, sq_ref, nf_ref, out_ref):
        s = sum_ref[0]
        q = sq_ref[0]
        for w in range(1, NWORK):
            s = s + sum_ref[w]
            q = q + sq_ref[w]
        nfv = nf_ref[...]
        mean = s / nfv
        msq = jnp.sum(mean * mean, axis=1, keepdims=True)
        sumsq = jnp.sum(q, axis=1, keepdims=True)
        nf1 = nfv[:, :1]
        var_total = jnp.sum((sumsq - nf1 * msq) / (nf1 - 1.0))
        norm = jnp.sqrt(msq)
        m = mean / jnp.maximum(norm, 1e-12)
        g = lax.dot_general(m, m, (((1,), (1,)), ((), ())))
        row = lax.broadcasted_iota(jnp.int32, (NSEG, NSEG), 0)
        col = lax.broadcasted_iota(jnp.int32, (NSEG, NSEG), 1)
        same = ((row // NC) == (col // NC)) & (row != col)
        sum_g = jnp.sum(jnp.where(same, g, 0.0))
        pairs_per_batch = NC * (NC - 1) // 2
        loss = 0.1 * (float(BS * pairs_per_batch) + 0.5 * sum_g) + var_total
        out_ref[...] = jnp.reshape(loss, (1, 1))

    out = pl.pallas_call(
        body,
        out_shape=jax.ShapeDtypeStruct((1, 1), jnp.float32),
    )(sum_parts, sq_parts, nf)
    return out.reshape(1)


def kernel(embeddings, cluster_sizes):
    cs = cluster_sizes.astype(jnp.int32)
    starts = jnp.concatenate(
        [jnp.zeros((BS, 1), jnp.int32), cs[:, :-1]], axis=1)
    ends = starts + cs
    meta = jnp.concatenate(
        [starts.reshape(-1), ends.reshape(-1),
         jnp.zeros((16,), jnp.int32)])
    nf = jnp.broadcast_to(
        cs.astype(jnp.float32).reshape(NSEG, 1), (NSEG, DIM))
    sum_parts, sq_parts = _sc_partial_sums(embeddings, meta)
    return _tc_finish(sum_parts, sq_parts.reshape(NWORK, NSEG, 16), nf)
